# baseline (speedup 1.0000x reference)
import jax
import jax.numpy as jnp
from jax import lax
from jax.experimental import pallas as pl
from jax.experimental.pallas import tpu as pltpu

_B, _M, _K, _E = 16, 8, 512, 1024


def _fused_kernel(sb_ref, pei_ref, cur_ref, bil_ref, vw_ref, allex_ref,
                  acts_ref, val_ref, q_ref):
    m = pl.program_id(1)

    @pl.when(m == 0)
    def _compute_query():
        q = jnp.dot(cur_ref[0], bil_ref[...],
                    preferred_element_type=jnp.float32)
        q_ref[...] = q
        v = jnp.dot(q, vw_ref[...],
                    preferred_element_type=jnp.float32) + sb_ref[1]
        val_ref[0] = jnp.broadcast_to(v, (_M, 1))

    q = q_ref[...]
    a = allex_ref[0, 0]
    scores = lax.dot_general(q, a, (((1,), (1,)), ((), ())),
                             preferred_element_type=jnp.float32)
    scores = scores + sb_ref[0]

    cand = lax.broadcasted_iota(jnp.int32, (_M, _K), 1)
    used = lax.broadcasted_iota(jnp.int32, (_M, _K), 0)
    peib = jnp.broadcast_to(pei_ref[0], (_M, _K))
    masked = jnp.any((cand == peib) & (used < m), axis=0, keepdims=True)
    acts_ref[0] = jnp.where(masked, -jnp.inf, scores)


def kernel(current_sample_encodings, example_encodings, all_example_encodings,
           policy_example_indices, bilinear, bias, value_w, value_b):
    del example_encodings
    sb = jnp.concatenate([bias, value_b])
    pei3 = policy_example_indices[:, :, None]
    cur3 = current_sample_encodings[:, None, :]

    acts, val = pl.pallas_call(
        _fused_kernel,
        grid=(_B, _M),
        in_specs=[
            pl.BlockSpec(memory_space=pltpu.SMEM),
            pl.BlockSpec((1, _M, 1), lambda b, m: (b, 0, 0)),
            pl.BlockSpec((1, 1, _E), lambda b, m: (b, 0, 0)),
            pl.BlockSpec((_E, _E), lambda b, m: (0, 0)),
            pl.BlockSpec((_E, 1), lambda b, m: (0, 0)),
            pl.BlockSpec((1, 1, _K, _E), lambda b, m: (b, m, 0, 0)),
        ],
        out_specs=[
            pl.BlockSpec((1, 1, _K), lambda b, m: (b * _M + m, 0, 0)),
            pl.BlockSpec((1, _M, 1), lambda b, m: (b, 0, 0)),
        ],
        out_shape=[
            jax.ShapeDtypeStruct((_B * _M, 1, _K), jnp.float32),
            jax.ShapeDtypeStruct((_B, _M, 1), jnp.float32),
        ],
        scratch_shapes=[pltpu.VMEM((1, _E), jnp.float32)],
        compiler_params=pltpu.CompilerParams(
            dimension_semantics=("arbitrary", "arbitrary")),
    )(sb, pei3, cur3, bilinear, value_w, all_example_encodings)

    activations_out = acts[:, 0, :]
    value_estimates = val[:, :, 0].reshape(-1)
    return activations_out, value_estimates


# speedup vs baseline: 1.6307x; 1.6307x over previous
import jax
import jax.numpy as jnp
from jax import lax
from jax.experimental import pallas as pl
from jax.experimental.pallas import tpu as pltpu

_B, _M, _K, _E = 16, 8, 512, 1024
_RM = 4


def _fused_kernel(sb_ref, pei_ref, cur_ref, bil_ref, vw_ref, allex_ref,
                  acts_ref, val_ref, q_ref):
    b = pl.program_id(0)
    mb = pl.program_id(1)

    @pl.when((b == 0) & (mb == 0))
    def _compute_queries():
        q = jnp.dot(cur_ref[...], bil_ref[...],
                    preferred_element_type=jnp.float32)
        q_ref[...] = q
        v = jnp.dot(q, vw_ref[...],
                    preferred_element_type=jnp.float32) + sb_ref[1]
        val_ref[...] = jnp.broadcast_to(v[:, :, None], (_B, _M, 1))

    q = q_ref[pl.ds(b, 1)]
    a = allex_ref[0].reshape(_RM * _K, _E)
    scores = lax.dot_general(q, a, (((1,), (1,)), ((), ())),
                             preferred_element_type=jnp.float32)
    scores = (scores + sb_ref[0]).reshape(_RM, _K)

    pei = pei_ref[0]
    cand = lax.broadcasted_iota(jnp.int32, (_M, _K), 1)
    hit = cand == jnp.broadcast_to(pei, (_M, _K))
    used = lax.broadcasted_iota(jnp.int32, (_M, _RM, _K), 0)
    row = lax.broadcasted_iota(jnp.int32, (_M, _RM, _K), 1) + mb * _RM
    masked = jnp.any(hit[:, None, :] & (used < row), axis=0)
    acts_ref[0] = jnp.where(masked, -jnp.inf, scores)


def kernel(current_sample_encodings, example_encodings, all_example_encodings,
           policy_example_indices, bilinear, bias, value_w, value_b):
    del example_encodings
    sb = jnp.concatenate([bias, value_b])
    pei3 = policy_example_indices[:, :, None]

    acts, val = pl.pallas_call(
        _fused_kernel,
        grid=(_B, _M // _RM),
        in_specs=[
            pl.BlockSpec(memory_space=pltpu.SMEM),
            pl.BlockSpec((1, _M, 1), lambda b, mb: (b, 0, 0)),
            pl.BlockSpec((_B, _E), lambda b, mb: (0, 0)),
            pl.BlockSpec((_E, _E), lambda b, mb: (0, 0)),
            pl.BlockSpec((_E, 1), lambda b, mb: (0, 0)),
            pl.BlockSpec((1, _RM, _K, _E),
                         lambda b, mb: (b, mb, 0, 0)),
        ],
        out_specs=[
            pl.BlockSpec((1, _RM, _K),
                         lambda b, mb: (b * (_M // _RM) + mb, 0, 0)),
            pl.BlockSpec((_B, _M, 1), lambda b, mb: (0, 0, 0)),
        ],
        out_shape=[
            jax.ShapeDtypeStruct((_B * (_M // _RM), _RM, _K), jnp.float32),
            jax.ShapeDtypeStruct((_B, _M, 1), jnp.float32),
        ],
        scratch_shapes=[pltpu.VMEM((_B, _E), jnp.float32)],
        compiler_params=pltpu.CompilerParams(
            dimension_semantics=("arbitrary", "arbitrary")),
    )(sb, pei3, current_sample_encodings, bilinear, value_w,
      all_example_encodings)

    activations_out = acts.reshape(_B * _M, _K)
    value_estimates = val[:, :, 0].reshape(-1)
    return activations_out, value_estimates


# speedup vs baseline: 1.6603x; 1.0182x over previous
import jax
import jax.numpy as jnp
from jax import lax
from jax.experimental import pallas as pl
from jax.experimental.pallas import tpu as pltpu

_B, _M, _K, _E = 16, 8, 512, 1024
_RM = 8


def _fused_kernel(sb_ref, pei_ref, cur_ref, bil_ref, vw_ref, allex_ref,
                  acts_ref, val_ref, q_ref):
    b = pl.program_id(0)
    mb = pl.program_id(1)

    @pl.when((b == 0) & (mb == 0))
    def _compute_queries():
        q = jnp.dot(cur_ref[...], bil_ref[...],
                    preferred_element_type=jnp.float32)
        q_ref[...] = q
        v = jnp.dot(q, vw_ref[...],
                    preferred_element_type=jnp.float32) + sb_ref[1]
        val_ref[...] = jnp.broadcast_to(v[:, :, None], (_B, _M, 1))

    q = q_ref[pl.ds(b, 1)]
    a = allex_ref[0].reshape(_RM * _K, _E)
    scores = lax.dot_general(q, a, (((1,), (1,)), ((), ())),
                             preferred_element_type=jnp.float32)
    scores = (scores + sb_ref[0]).reshape(_RM, _K)

    pei = pei_ref[0]
    cand = lax.broadcasted_iota(jnp.int32, (_M, _K), 1)
    hit = cand == jnp.broadcast_to(pei, (_M, _K))
    used = lax.broadcasted_iota(jnp.int32, (_M, _RM, _K), 0)
    row = lax.broadcasted_iota(jnp.int32, (_M, _RM, _K), 1) + mb * _RM
    masked = jnp.any(hit[:, None, :] & (used < row), axis=0)
    acts_ref[0] = jnp.where(masked, -jnp.inf, scores)


def kernel(current_sample_encodings, example_encodings, all_example_encodings,
           policy_example_indices, bilinear, bias, value_w, value_b):
    del example_encodings
    sb = jnp.concatenate([bias, value_b])
    pei3 = policy_example_indices[:, :, None]

    acts, val = pl.pallas_call(
        _fused_kernel,
        grid=(_B, _M // _RM),
        in_specs=[
            pl.BlockSpec(memory_space=pltpu.SMEM),
            pl.BlockSpec((1, _M, 1), lambda b, mb: (b, 0, 0)),
            pl.BlockSpec((_B, _E), lambda b, mb: (0, 0)),
            pl.BlockSpec((_E, _E), lambda b, mb: (0, 0)),
            pl.BlockSpec((_E, 1), lambda b, mb: (0, 0)),
            pl.BlockSpec((1, _RM, _K, _E),
                         lambda b, mb: (b, mb, 0, 0)),
        ],
        out_specs=[
            pl.BlockSpec((1, _RM, _K),
                         lambda b, mb: (b * (_M // _RM) + mb, 0, 0)),
            pl.BlockSpec((_B, _M, 1), lambda b, mb: (0, 0, 0)),
        ],
        out_shape=[
            jax.ShapeDtypeStruct((_B * (_M // _RM), _RM, _K), jnp.float32),
            jax.ShapeDtypeStruct((_B, _M, 1), jnp.float32),
        ],
        scratch_shapes=[pltpu.VMEM((_B, _E), jnp.float32)],
        compiler_params=pltpu.CompilerParams(
            dimension_semantics=("arbitrary", "arbitrary")),
    )(sb, pei3, current_sample_encodings, bilinear, value_w,
      all_example_encodings)

    activations_out = acts.reshape(_B * _M, _K)
    value_estimates = val[:, :, 0].reshape(-1)
    return activations_out, value_estimates
